# Initial kernel scaffold; baseline (speedup 1.0000x reference)
import jax
import jax.numpy as jnp
from jax.experimental import pallas as pl


def kernel(inputs, w1, b1, g1, be1, w2, b2, g2, be2, w3, b3, mlp_w, mlp_b):
    raise NotImplementedError("write your pallas kernel here")


# speedup vs baseline: 9.1350x; 9.1350x over previous
import functools

import jax
import jax.numpy as jnp
from jax.experimental import pallas as pl
from jax.experimental.pallas import tpu as pltpu
from jax.experimental.pallas import tpu_sc as plsc

B, C, N, K = 8, 64, 2048, 16
BN_ROWS = B * N
BNK = B * N * K
TN = 256
TR = 128
TRK = TR * K
G = BNK // TRK
NEG_BIG = 3.0e38


def _p1_body(x_ref, w1a_ref, wc_ref, idx_ref, z_ref, c1_ref):
    b = pl.program_id(0)
    i = pl.program_id(1)
    xall_t = x_ref[0]

    @pl.when(i == 0)
    def _():
        zval = jax.lax.dot_general(
            xall_t, w1a_ref[...], (((0,), (1,)), ((), ())),
            preferred_element_type=jnp.float32)
        z_ref[0] = jnp.concatenate(
            [zval, jnp.zeros((N, C), jnp.float32)], axis=1)
        c1_ref[0] = jax.lax.dot_general(
            xall_t, wc_ref[...], (((0,), (1,)), ((), ())),
            preferred_element_type=jnp.float32)

    xt_t = x_ref[0, :, pl.ds(i * TN, TN)]
    prod = jax.lax.dot_general(
        xt_t, xall_t, (((0,), (0,)), ((), ())),
        preferred_element_type=jnp.float32)
    nall = jnp.sum(xall_t * xall_t, axis=0, keepdims=True)
    dist = nall - 2.0 * prod

    col = jax.lax.broadcasted_iota(jnp.int32, (TN, N), 1)
    picks = []
    for _k in range(K):
        m = jnp.min(dist, axis=1, keepdims=True)
        cand = jnp.where(dist <= m, col, N)
        ik = jnp.min(cand, axis=1, keepdims=True)
        dist = jnp.where(col == ik, NEG_BIG, dist)
        picks.append(ik)
    idx_ref[0] = jnp.concatenate(picks, axis=1) + b * N


def _p1(x, w1a, wc):
    return pl.pallas_call(
        _p1_body,
        grid=(B, N // TN),
        in_specs=[
            pl.BlockSpec((1, C, N), lambda b, i: (b, 0, 0)),
            pl.BlockSpec((C, C), lambda b, i: (0, 0)),
            pl.BlockSpec((C, C), lambda b, i: (0, 0)),
        ],
        out_specs=[
            pl.BlockSpec((1, TN, K), lambda b, i: (b, i, 0)),
            pl.BlockSpec((1, N, 2 * C), lambda b, i: (b, 0, 0)),
            pl.BlockSpec((1, N, C), lambda b, i: (b, 0, 0)),
        ],
        out_shape=[
            jax.ShapeDtypeStruct((B, N, K), jnp.int32),
            jax.ShapeDtypeStruct((B, N, 2 * C), jnp.float32),
            jax.ShapeDtypeStruct((B, N, C), jnp.float32),
        ],
        compiler_params=pltpu.CompilerParams(
            dimension_semantics=("parallel", "arbitrary")),
    )(x, w1a, wc)


_NW = 32
_GCH = 256
_GCHUNKS = BNK // (_NW * _GCH)


def _sc_gather(z_flat, idx_flat):
    mesh = plsc.VectorSubcoreMesh(core_axis_name="c", subcore_axis_name="s")

    @functools.partial(
        pl.kernel, mesh=mesh,
        out_type=jax.ShapeDtypeStruct((BNK, 2 * C), jnp.float32),
        scratch_types=[
            pltpu.VMEM((_GCH,), jnp.int32),
            pltpu.VMEM((_GCH, 2 * C), jnp.float32),
            pltpu.SemaphoreType.DMA,
        ],
    )
    def _gather_kernel(table_hbm, idx_hbm, out_hbm, idx_v, rows_v, sem):
        wid = jax.lax.axis_index("s") * 2 + jax.lax.axis_index("c")

        @pl.loop(0, _GCHUNKS)
        def _(j):
            base = (wid * _GCHUNKS + j) * _GCH
            pltpu.sync_copy(idx_hbm.at[pl.ds(base, _GCH)], idx_v)
            pltpu.async_copy(table_hbm.at[idx_v], rows_v, sem).wait()
            pltpu.sync_copy(rows_v, out_hbm.at[pl.ds(base, _GCH)])

    return _gather_kernel(z_flat, idx_flat)


def _expand_c1(c1_ref):
    c1b = c1_ref[...]
    return jnp.broadcast_to(c1b[:, None, :], (TR, K, C)).reshape(TRK, C)


def _stats_rows(h):
    s = jnp.sum(h, axis=0, keepdims=True)
    ss = jnp.sum(h * h, axis=0, keepdims=True)
    return jnp.concatenate(
        [s, ss, jnp.zeros((6, h.shape[1]), jnp.float32)], axis=0)


def _p3_body(zg_ref, c1_ref, st_ref):
    h = _expand_c1(c1_ref) - zg_ref[:, :C]
    st_ref[0] = _stats_rows(h)


def _p3(zg, c1_flat):
    return pl.pallas_call(
        _p3_body,
        grid=(G,),
        in_specs=[
            pl.BlockSpec((TRK, 2 * C), lambda g: (g, 0)),
            pl.BlockSpec((TR, C), lambda g: (g, 0)),
        ],
        out_specs=pl.BlockSpec((1, 8, C), lambda g: (g, 0, 0)),
        out_shape=jax.ShapeDtypeStruct((G, 8, C), jnp.float32),
        compiler_params=pltpu.CompilerParams(
            dimension_semantics=("parallel",)),
    )(zg, c1_flat)


def _lrelu(x):
    return jnp.where(x > 0, x, 0.2 * x)


def _p4_body(zg_ref, c1_ref, st1_ref, w2t_ref, h2_ref, st_ref):
    h1p = _expand_c1(c1_ref) - zg_ref[:, :C]
    h1 = _lrelu(st1_ref[0:1, :] * h1p + st1_ref[1:2, :])
    h2p = jnp.dot(h1, w2t_ref[...], preferred_element_type=jnp.float32)
    h2_ref[...] = h2p
    st_ref[0] = _stats_rows(h2p)


def _p4(zg, c1_flat, st1, w2t):
    return pl.pallas_call(
        _p4_body,
        grid=(G,),
        in_specs=[
            pl.BlockSpec((TRK, 2 * C), lambda g: (g, 0)),
            pl.BlockSpec((TR, C), lambda g: (g, 0)),
            pl.BlockSpec((8, C), lambda g: (0, 0)),
            pl.BlockSpec((C, C), lambda g: (0, 0)),
        ],
        out_specs=[
            pl.BlockSpec((TRK, C), lambda g: (g, 0)),
            pl.BlockSpec((1, 8, C), lambda g: (g, 0, 0)),
        ],
        out_shape=[
            jax.ShapeDtypeStruct((BNK, C), jnp.float32),
            jax.ShapeDtypeStruct((G, 8, C), jnp.float32),
        ],
        compiler_params=pltpu.CompilerParams(
            dimension_semantics=("parallel",)),
    )(zg, c1_flat, st1, w2t)


def _p5_body(h2_ref, st2_ref, w3t_ref, mlpt_ref, bias_ref, out_ref):
    h2 = _lrelu(st2_ref[0:1, :] * h2_ref[...] + st2_ref[1:2, :])
    h3 = jnp.dot(h2, w3t_ref[...], preferred_element_type=jnp.float32)
    h3m = jnp.max(h3.reshape(TR, K, 2 * C), axis=1) + bias_ref[0:1, :]
    h4 = jnp.dot(h3m, mlpt_ref[...], preferred_element_type=jnp.float32)
    out_ref[...] = h4 + bias_ref[1:2, :]


def _p5(h2, st2, w3t, mlpt, bias):
    return pl.pallas_call(
        _p5_body,
        grid=(G,),
        in_specs=[
            pl.BlockSpec((TRK, C), lambda g: (g, 0)),
            pl.BlockSpec((8, C), lambda g: (0, 0)),
            pl.BlockSpec((C, 2 * C), lambda g: (0, 0)),
            pl.BlockSpec((2 * C, 2 * C), lambda g: (0, 0)),
            pl.BlockSpec((8, 2 * C), lambda g: (0, 0)),
        ],
        out_specs=pl.BlockSpec((TR, 2 * C), lambda g: (g, 0)),
        out_shape=jax.ShapeDtypeStruct((BN_ROWS, 2 * C), jnp.float32),
        compiler_params=pltpu.CompilerParams(
            dimension_semantics=("parallel",)),
    )(h2, st2, w3t, mlpt, bias)


def _bn_coeffs(partials, g, be):
    sums = jnp.sum(partials, axis=0)
    mean = sums[0] / BNK
    var = sums[1] / BNK - mean * mean
    s = g / jnp.sqrt(var + 1e-5)
    t = be - mean * s
    pad = jnp.zeros((6, s.shape[0]), jnp.float32)
    return jnp.concatenate([s[None], t[None], pad], axis=0)


def kernel(inputs, w1, b1, g1, be1, w2, b2, g2, be2, w3, b3, mlp_w, mlp_b):
    w1a = w1[:, :C]
    wc = w1a + w1[:, C:]

    idx, z, c1 = _p1(inputs, w1a, wc)
    zg = _sc_gather(z.reshape(BN_ROWS, 2 * C), idx.reshape(BNK))
    c1_flat = c1.reshape(BN_ROWS, C)

    st1 = _bn_coeffs(_p3(zg, c1_flat), g1, be1)
    h2, st2_part = _p4(zg, c1_flat, st1, w2.T)
    st2 = _bn_coeffs(st2_part, g2, be2)

    bias = jnp.concatenate(
        [b3[None], mlp_b[None], jnp.zeros((6, 2 * C), jnp.float32)], axis=0)
    h4 = _p5(h2, st2, w3.T, mlp_w.T, bias)

    out = h4.reshape(B, N, 2, C).transpose(0, 3, 1, 2).reshape(B, C, 2 * N)
    return out
